# Initial kernel scaffold; baseline (speedup 1.0000x reference)
import jax
import jax.numpy as jnp
from jax.experimental import pallas as pl


def kernel(encoded, memory, W, b):
    raise NotImplementedError("write your pallas kernel here")


# speedup vs baseline: 4.7700x; 4.7700x over previous
import functools

import jax
import jax.numpy as jnp
from jax import lax
from jax.experimental import pallas as pl
from jax.experimental.pallas import tpu as pltpu
from jax.experimental.pallas import tpu_sc as plsc

_TOPK = 8
_NEG = float("-inf")


def _topk_body(enc_ref, mem_ref, idx_out_ref, rv_ref, ri_ref, *, chunk, n_chunks, kv_len):
    b = pl.program_id(0)
    c = pl.program_id(1)

    @pl.when(c == 0)
    def _init():
        rv_ref[...] = jnp.full((_TOPK, 128), _NEG, jnp.float32)
        ri_ref[...] = jnp.zeros((_TOPK, 128), jnp.int32)

    enc = enc_ref[0]
    mem = mem_ref[0]

    en = jnp.sum(enc * enc, axis=1, keepdims=True)
    encn = enc / jnp.maximum(jnp.sqrt(en), 1e-12)

    sq = jnp.sum(mem * mem, axis=1)
    scale = 1.0 / jnp.maximum(jnp.sqrt(sq), 1e-12)

    sim = lax.dot_general(encn, mem, (((1,), (1,)), ((), ())),
                          precision=lax.Precision.HIGHEST,
                          preferred_element_type=jnp.float32)
    sim = sim * scale[None, :]

    base = b * kv_len + c * chunk
    chunk_idx = base + lax.broadcasted_iota(jnp.int32, (_TOPK, chunk), 1)

    ext_v = jnp.concatenate([rv_ref[...], sim], axis=1)
    ext_i = jnp.concatenate([ri_ref[...], chunk_idx], axis=1)

    width = 128 + chunk
    pos = lax.broadcasted_iota(jnp.int32, (_TOPK, width), 1)
    big = jnp.int32(2**30)

    vals, idxs = [], []
    for _ in range(_TOPK):
        m = jnp.max(ext_v, axis=1, keepdims=True)
        eq = ext_v == m
        first = jnp.min(jnp.where(eq, pos, big), axis=1, keepdims=True)
        onehot = pos == first
        vals.append(m[:, 0])
        idxs.append(jnp.sum(jnp.where(onehot, ext_i, 0), axis=1))
        ext_v = jnp.where(onehot, _NEG, ext_v)

    new_v = jnp.stack(vals, axis=1)
    new_i = jnp.stack(idxs, axis=1)
    rv_ref[...] = jnp.concatenate(
        [new_v, jnp.full((_TOPK, 128 - _TOPK), _NEG, jnp.float32)], axis=1)
    ri_ref[...] = jnp.concatenate(
        [new_i, jnp.zeros((_TOPK, 128 - _TOPK), jnp.int32)], axis=1)

    @pl.when(c == n_chunks - 1)
    def _emit():
        idx_out_ref[0] = ri_ref[...]


def _topk_indices(encoded, memory, chunk=4096):
    B, L, D = encoded.shape
    kv_len = memory.shape[1]
    n_chunks = kv_len // chunk
    body = functools.partial(_topk_body, chunk=chunk, n_chunks=n_chunks,
                             kv_len=kv_len)
    return pl.pallas_call(
        body,
        grid=(B, n_chunks),
        in_specs=[
            pl.BlockSpec((1, L, D), lambda b, c: (b, 0, 0)),
            pl.BlockSpec((1, chunk, D), lambda b, c: (b, c, 0)),
        ],
        out_specs=pl.BlockSpec((1, L, 128), lambda b, c: (b, 0, 0)),
        out_shape=jax.ShapeDtypeStruct((B, L, 128), jnp.int32),
        scratch_shapes=[
            pltpu.VMEM((_TOPK, 128), jnp.float32),
            pltpu.VMEM((_TOPK, 128), jnp.int32),
        ],
    )(encoded, memory)


def _gather_mean(mem_flat, idx_flat):
    n_rows, D = mem_flat.shape
    n_idx = idx_flat.shape[0]
    info = plsc.get_sparse_core_info()
    nc, ns, nl = info.num_cores, info.num_subcores, info.num_lanes
    nw = nc * ns
    per_w = n_idx // nw
    out_rows = n_idx // _TOPK
    out_per_w = out_rows // nw

    mesh = plsc.VectorSubcoreMesh(core_axis_name="c", subcore_axis_name="s")

    @functools.partial(
        pl.kernel, mesh=mesh,
        out_type=jax.ShapeDtypeStruct((out_rows, D), jnp.float32),
        scratch_types=[
            pltpu.VMEM((per_w,), jnp.int32),
            pltpu.VMEM((per_w, D), jnp.float32),
            pltpu.VMEM((out_per_w, D), jnp.float32),
            pltpu.SemaphoreType.DMA,
        ],
    )
    def k(mem_hbm, idx_hbm, out_hbm, idx_v, rows_v, acc_v, sem):
        wid = lax.axis_index("s") * nc + lax.axis_index("c")
        base = wid * per_w
        pltpu.sync_copy(idx_hbm.at[pl.ds(base, per_w)], idx_v)
        pltpu.async_copy(mem_hbm.at[idx_v], rows_v, sem).wait()
        inv = jnp.float32(1.0 / _TOPK)
        for r in range(out_per_w):
            for t in range(D // nl):
                acc = jnp.zeros((nl,), jnp.float32)
                for j in range(_TOPK):
                    acc = acc + rows_v[r * _TOPK + j, pl.ds(t * nl, nl)]
                acc_v[r, pl.ds(t * nl, nl)] = acc * inv
        pltpu.sync_copy(acc_v, out_hbm.at[pl.ds(wid * out_per_w, out_per_w)])

    return k(mem_flat, idx_flat)


def _affine_body(e_ref, m_ref, w_ref, b_ref, o_ref):
    x = e_ref[...] + m_ref[...]
    y = lax.dot_general(x, w_ref[...], (((1,), (1,)), ((), ())),
                        preferred_element_type=jnp.float32)
    o_ref[...] = y + b_ref[...]


def _affine(enc_flat, matched, W, b):
    n, D = enc_flat.shape
    return pl.pallas_call(
        _affine_body,
        out_shape=jax.ShapeDtypeStruct((n, D), jnp.float32),
    )(enc_flat, matched, W, b.reshape(1, D))


def kernel(encoded, memory, W, b):
    B, L, D = encoded.shape
    kv_len = memory.shape[1]
    idx_pad = _topk_indices(encoded, memory)
    idx_flat = idx_pad[:, :, :_TOPK].reshape(-1)
    matched = _gather_mean(memory.reshape(B * kv_len, D), idx_flat)
    out = _affine(encoded.reshape(B * L, D), matched, W, b)
    return out.reshape(B, L, D)


# speedup vs baseline: 5.2747x; 1.1058x over previous
import functools

import jax
import jax.numpy as jnp
from jax import lax
from jax.experimental import pallas as pl
from jax.experimental.pallas import tpu as pltpu
from jax.experimental.pallas import tpu_sc as plsc

_TOPK = 8
_CANDS = 16
_NEG = float("-inf")
_SHIFT = 8192
_FMASK = _SHIFT - 1


def _filter_body(enc_ref, mem_ref, idx_out_ref, rk_ref, ri_ref, *,
                 chunk, n_chunks, kv_len):
    b = pl.program_id(0)
    c = pl.program_id(1)

    @pl.when(c == 0)
    def _init():
        rk_ref[...] = jnp.zeros((_TOPK, 128), jnp.int32)
        ri_ref[...] = jnp.zeros((_TOPK, 128), jnp.int32)

    enc = enc_ref[0]
    mem = mem_ref[0]

    en = jnp.sum(enc * enc, axis=1, keepdims=True)
    encn = enc / jnp.maximum(jnp.sqrt(en), 1e-12)

    sq = jnp.sum(mem * mem, axis=1)
    scale = 1.0 / jnp.maximum(jnp.sqrt(sq), 1e-12)

    sim = lax.dot_general(encn, mem, (((1,), (1,)), ((), ())),
                          preferred_element_type=jnp.float32)
    sim = sim * scale[None, :]

    t = ((sim + 1.25) * 65536.0).astype(jnp.int32)
    local = lax.broadcasted_iota(jnp.int32, (_TOPK, chunk), 1)
    keys = t * _SHIFT + (chunk - 1 - local)

    ext = jnp.concatenate([rk_ref[...], keys], axis=1)
    lane128 = lax.broadcasted_iota(jnp.int32, (_TOPK, 128), 1)
    base = b * kv_len + c * chunk
    run_tag0 = chunk + _CANDS - 1

    new_k, new_i = [], []
    ri = ri_ref[...]
    for r in range(_CANDS):
        m = jnp.max(ext, axis=1, keepdims=True)
        tag = m & _FMASK
        is_run = tag >= chunk
        slot = run_tag0 - tag
        localpos = (chunk - 1) - tag
        abs_run = jnp.sum(jnp.where(lane128 == slot, ri, 0),
                          axis=1, keepdims=True)
        abs_idx = jnp.where(is_run, abs_run, base + localpos)
        new_k.append((m - tag + (run_tag0 - r))[:, 0])
        new_i.append(abs_idx[:, 0])
        ext = jnp.where(ext == m, 0, ext)

    nk = jnp.stack(new_k, axis=1)
    ni = jnp.stack(new_i, axis=1)
    zpad = jnp.zeros((_TOPK, 128 - _CANDS), jnp.int32)
    rk_ref[...] = jnp.concatenate([nk, zpad], axis=1)
    ri_ref[...] = jnp.concatenate([ni, zpad], axis=1)

    @pl.when(c == n_chunks - 1)
    def _emit():
        idx_out_ref[0] = ri_ref[...]


def _filter_indices(encoded, memory, chunk=4096):
    B, L, D = encoded.shape
    kv_len = memory.shape[1]
    n_chunks = kv_len // chunk
    body = functools.partial(_filter_body, chunk=chunk, n_chunks=n_chunks,
                             kv_len=kv_len)
    return pl.pallas_call(
        body,
        grid=(B, n_chunks),
        in_specs=[
            pl.BlockSpec((1, L, D), lambda b, c: (b, 0, 0)),
            pl.BlockSpec((1, chunk, D), lambda b, c: (b, c, 0)),
        ],
        out_specs=pl.BlockSpec((1, L, 128), lambda b, c: (b, 0, 0)),
        out_shape=jax.ShapeDtypeStruct((B, L, 128), jnp.int32),
        scratch_shapes=[
            pltpu.VMEM((_TOPK, 128), jnp.int32),
            pltpu.VMEM((_TOPK, 128), jnp.int32),
        ],
    )(encoded, memory)


def _gather_rows(mem_flat, idx_flat):
    n_rows, D = mem_flat.shape
    n_idx = idx_flat.shape[0]
    info = plsc.get_sparse_core_info()
    nc, ns = info.num_cores, info.num_subcores
    nw = nc * ns
    per_w = n_idx // nw

    mesh = plsc.VectorSubcoreMesh(core_axis_name="c", subcore_axis_name="s")

    @functools.partial(
        pl.kernel, mesh=mesh,
        out_type=jax.ShapeDtypeStruct((n_idx, D), jnp.float32),
        scratch_types=[
            pltpu.VMEM((per_w,), jnp.int32),
            pltpu.VMEM((per_w, D), jnp.float32),
            pltpu.SemaphoreType.DMA,
        ],
    )
    def k(mem_hbm, idx_hbm, out_hbm, idx_v, rows_v, sem):
        wid = lax.axis_index("s") * nc + lax.axis_index("c")
        base = wid * per_w
        pltpu.sync_copy(idx_hbm.at[pl.ds(base, per_w)], idx_v)
        pltpu.async_copy(mem_hbm.at[idx_v], rows_v, sem).wait()
        pltpu.sync_copy(rows_v, out_hbm.at[pl.ds(base, per_w)])

    return k(mem_flat, idx_flat)


def _rescore_body(enc_ref, rows_ref, w_ref, b_ref, out_ref, *, nq):
    rows = rows_ref[...]
    n2 = jnp.sum(rows * rows, axis=1)
    norm = jnp.where(n2 == 0.0, 0.0, n2 * lax.rsqrt(n2))
    rown = rows / jnp.maximum(norm, 1e-12)[:, None]

    enc = enc_ref[...]
    en2 = jnp.sum(enc * enc, axis=1)
    enorm = jnp.where(en2 == 0.0, 0.0, en2 * lax.rsqrt(en2))
    encn = enc / jnp.maximum(enorm, 1e-12)[:, None]

    rb = rown.astype(jnp.bfloat16).astype(jnp.float32)
    eb = encn.astype(jnp.bfloat16).astype(jnp.float32)
    prod = rb.reshape(nq, _CANDS, 128) * eb.reshape(nq, 1, 128)
    simx = jnp.sum(prod, axis=2)

    pos = lax.broadcasted_iota(jnp.int32, (nq, _CANDS), 1)
    wsel = jnp.zeros((nq, _CANDS), jnp.float32)
    ext = simx
    for _ in range(_TOPK):
        m = jnp.max(ext, axis=1, keepdims=True)
        eq = ext == m
        first = jnp.min(jnp.where(eq, pos, _CANDS), axis=1, keepdims=True)
        onehot = pos == first
        wsel = wsel + jnp.where(onehot, 1.0, 0.0)
        ext = jnp.where(onehot, _NEG, ext)

    picked = rows.reshape(nq, _CANDS, 128) * wsel[:, :, None]
    matched = jnp.sum(picked, axis=1) * (1.0 / _TOPK)

    x = enc + matched
    y = lax.dot_general(x, w_ref[...], (((1,), (1,)), ((), ())),
                        preferred_element_type=jnp.float32)
    out_ref[...] = y + b_ref[...]


def _rescore(enc_flat, rows, W, b):
    nq, D = enc_flat.shape
    return pl.pallas_call(
        functools.partial(_rescore_body, nq=nq),
        out_shape=jax.ShapeDtypeStruct((nq, D), jnp.float32),
    )(enc_flat, rows, W, b.reshape(1, D))


def kernel(encoded, memory, W, b):
    B, L, D = encoded.shape
    kv_len = memory.shape[1]
    idx_pad = _filter_indices(encoded, memory)
    idx_flat = idx_pad[:, :, :_CANDS].reshape(-1)
    rows = _gather_rows(memory.reshape(B * kv_len, D), idx_flat)
    out = _rescore(encoded.reshape(B * L, D), rows, W, b)
    return out.reshape(B, L, D)


# speedup vs baseline: 7.0575x; 1.3380x over previous
import functools

import jax
import jax.numpy as jnp
from jax import lax
from jax.experimental import pallas as pl
from jax.experimental.pallas import tpu as pltpu
from jax.experimental.pallas import tpu_sc as plsc

_TOPK = 8
_CANDS = 12
_NEG = float("-inf")
_SHIFT = 8192
_FMASK = _SHIFT - 1


def _filter_body(enc_ref, mem_ref, idx_out_ref, rk_ref, ri_ref, *,
                 chunk, n_chunks, kv_len):
    b = pl.program_id(0)
    c = pl.program_id(1)

    @pl.when(c == 0)
    def _init():
        rk_ref[...] = jnp.zeros((_TOPK, 128), jnp.int32)
        ri_ref[...] = jnp.zeros((_TOPK, 128), jnp.int32)

    enc = enc_ref[0]
    mem = mem_ref[0]

    en = jnp.sum(enc * enc, axis=1, keepdims=True)
    encn = enc / jnp.maximum(jnp.sqrt(en), 1e-12)

    ones = jnp.ones((1, 128), jnp.float32)
    n2 = lax.dot_general(ones, mem * mem, (((1,), (1,)), ((), ())),
                         preferred_element_type=jnp.float32)
    scale = lax.rsqrt(jnp.maximum(n2, 1e-24))

    sim = lax.dot_general(encn, mem, (((1,), (1,)), ((), ())),
                          preferred_element_type=jnp.float32)
    sim = sim * scale

    t = ((sim + 1.25) * 65536.0).astype(jnp.int32)
    local = lax.broadcasted_iota(jnp.int32, (_TOPK, chunk), 1)
    keys = t * _SHIFT + (chunk - 1 - local)

    ext = jnp.concatenate([rk_ref[...], keys], axis=1)
    lane128 = lax.broadcasted_iota(jnp.int32, (_TOPK, 128), 1)
    base = b * kv_len + c * chunk
    run_tag0 = chunk + _CANDS - 1

    new_k, new_i = [], []
    ri = ri_ref[...]
    for r in range(_CANDS):
        m = jnp.max(ext, axis=1, keepdims=True)
        tag = m & _FMASK
        is_run = tag >= chunk
        slot = run_tag0 - tag
        localpos = (chunk - 1) - tag
        abs_run = jnp.sum(jnp.where(lane128 == slot, ri, 0),
                          axis=1, keepdims=True)
        abs_idx = jnp.where(is_run, abs_run, base + localpos)
        new_k.append((m - tag + (run_tag0 - r))[:, 0])
        new_i.append(abs_idx[:, 0])
        ext = jnp.where(ext == m, 0, ext)

    nk = jnp.stack(new_k, axis=1)
    ni = jnp.stack(new_i, axis=1)
    zpad = jnp.zeros((_TOPK, 128 - _CANDS), jnp.int32)
    rk_ref[...] = jnp.concatenate([nk, zpad], axis=1)
    ri_ref[...] = jnp.concatenate([ni, zpad], axis=1)

    @pl.when(c == n_chunks - 1)
    def _emit():
        idx_out_ref[0] = ri_ref[...]


def _filter_indices(encoded, memory, chunk=4096):
    B, L, D = encoded.shape
    kv_len = memory.shape[1]
    n_chunks = kv_len // chunk
    body = functools.partial(_filter_body, chunk=chunk, n_chunks=n_chunks,
                             kv_len=kv_len)
    return pl.pallas_call(
        body,
        grid=(B, n_chunks),
        in_specs=[
            pl.BlockSpec((1, L, D), lambda b, c: (b, 0, 0)),
            pl.BlockSpec((1, chunk, D), lambda b, c: (b, c, 0)),
        ],
        out_specs=pl.BlockSpec((1, L, 128), lambda b, c: (b, 0, 0)),
        out_shape=jax.ShapeDtypeStruct((B, L, 128), jnp.int32),
        scratch_shapes=[
            pltpu.VMEM((_TOPK, 128), jnp.int32),
            pltpu.VMEM((_TOPK, 128), jnp.int32),
        ],
    )(encoded, memory)


def _gather_rows(mem_flat, idx_flat):
    n_rows, D = mem_flat.shape
    n_idx = idx_flat.shape[0]
    info = plsc.get_sparse_core_info()
    nc, ns = info.num_cores, info.num_subcores
    nw = nc * ns
    per_w = n_idx // nw

    mesh = plsc.VectorSubcoreMesh(core_axis_name="c", subcore_axis_name="s")

    @functools.partial(
        pl.kernel, mesh=mesh,
        out_type=jax.ShapeDtypeStruct((n_idx, D), jnp.float32),
        scratch_types=[
            pltpu.VMEM((per_w,), jnp.int32),
            pltpu.VMEM((per_w, D), jnp.float32),
            pltpu.SemaphoreType.DMA,
        ],
    )
    def k(mem_hbm, idx_hbm, out_hbm, idx_v, rows_v, sem):
        wid = lax.axis_index("s") * nc + lax.axis_index("c")
        base = wid * per_w
        pltpu.sync_copy(idx_hbm.at[pl.ds(base, per_w)], idx_v)
        pltpu.async_copy(mem_hbm.at[idx_v], rows_v, sem).wait()
        pltpu.sync_copy(rows_v, out_hbm.at[pl.ds(base, per_w)])

    return k(mem_flat, idx_flat)


def _rescore_body(enc_ref, rows_ref, w_ref, b_ref, out_ref, *, nq):
    rows = rows_ref[...]
    n2 = jnp.sum(rows * rows, axis=1)
    norm = jnp.where(n2 == 0.0, 0.0, n2 * lax.rsqrt(n2))
    rown = rows / jnp.maximum(norm, 1e-12)[:, None]

    enc = enc_ref[...]
    en2 = jnp.sum(enc * enc, axis=1)
    enorm = jnp.where(en2 == 0.0, 0.0, en2 * lax.rsqrt(en2))
    encn = enc / jnp.maximum(enorm, 1e-12)[:, None]

    rb = rown.astype(jnp.bfloat16).astype(jnp.float32)
    eb = encn.astype(jnp.bfloat16).astype(jnp.float32)
    prod = rb.reshape(nq, _CANDS, 128) * eb.reshape(nq, 1, 128)
    simx = jnp.sum(prod, axis=2)

    pos = lax.broadcasted_iota(jnp.int32, (nq, _CANDS), 1)
    wsel = jnp.zeros((nq, _CANDS), jnp.float32)
    ext = simx
    for _ in range(_TOPK):
        m = jnp.max(ext, axis=1, keepdims=True)
        eq = ext == m
        first = jnp.min(jnp.where(eq, pos, _CANDS), axis=1, keepdims=True)
        onehot = pos == first
        wsel = wsel + jnp.where(onehot, 1.0, 0.0)
        ext = jnp.where(onehot, _NEG, ext)

    picked = rows.reshape(nq, _CANDS, 128) * wsel[:, :, None]
    matched = jnp.sum(picked, axis=1) * (1.0 / _TOPK)

    x = enc + matched
    y = lax.dot_general(x, w_ref[...], (((1,), (1,)), ((), ())),
                        preferred_element_type=jnp.float32)
    out_ref[...] = y + b_ref[...]


def _rescore(enc_flat, rows, W, b):
    nq, D = enc_flat.shape
    return pl.pallas_call(
        functools.partial(_rescore_body, nq=nq),
        out_shape=jax.ShapeDtypeStruct((nq, D), jnp.float32),
    )(enc_flat, rows, W, b.reshape(1, D))


def kernel(encoded, memory, W, b):
    B, L, D = encoded.shape
    kv_len = memory.shape[1]
    idx_pad = _filter_indices(encoded, memory)
    idx_flat = idx_pad[:, :, :_CANDS].reshape(-1)
    rows = _gather_rows(memory.reshape(B * kv_len, D), idx_flat)
    out = _rescore(encoded.reshape(B * L, D), rows, W, b)
    return out.reshape(B, L, D)


# speedup vs baseline: 16.1161x; 2.2836x over previous
import functools

import jax
import jax.numpy as jnp
from jax import lax
from jax.experimental import pallas as pl
from jax.experimental.pallas import tpu as pltpu
from jax.experimental.pallas import tpu_sc as plsc

_TOPK = 8
_CANDS = 12
_NEG = float("-inf")
_SHIFT = 8192
_FMASK = _SHIFT - 1


def _filter_body(enc_ref, mem_ref, idx_out_ref, rk_ref, ri_ref, *,
                 chunk, n_chunks, kv_len, bg):
    g = pl.program_id(0)
    c = pl.program_id(1)
    rows = bg * _TOPK

    @pl.when(c == 0)
    def _init():
        rk_ref[...] = jnp.zeros((rows, 128), jnp.int32)
        ri_ref[...] = jnp.zeros((rows, 128), jnp.int32)

    ones = jnp.ones((1, 128), jnp.float32)
    keys_parts = []
    for i in range(bg):
        enc = enc_ref[i]
        mem = mem_ref[i]
        en = jnp.sum(enc * enc, axis=1, keepdims=True)
        encn = enc / jnp.maximum(jnp.sqrt(en), 1e-12)
        n2 = lax.dot_general(ones, mem * mem, (((1,), (1,)), ((), ())),
                             preferred_element_type=jnp.float32)
        scale = lax.rsqrt(jnp.maximum(n2, 1e-24))
        sim = lax.dot_general(encn, mem, (((1,), (1,)), ((), ())),
                              preferred_element_type=jnp.float32)
        sim = sim * scale
        t = ((sim + 1.25) * 65536.0).astype(jnp.int32)
        local = lax.broadcasted_iota(jnp.int32, (_TOPK, chunk), 1)
        keys_parts.append(t * _SHIFT + (chunk - 1 - local))

    keys = jnp.concatenate(keys_parts, axis=0)
    ext = jnp.concatenate([rk_ref[...], keys], axis=1)
    lane128 = lax.broadcasted_iota(jnp.int32, (rows, 128), 1)
    sub = lax.broadcasted_iota(jnp.int32, (rows, 1), 0) // _TOPK
    base = (g * bg + sub) * kv_len + c * chunk
    run_tag0 = chunk + _CANDS - 1

    new_k, new_i = [], []
    ri = ri_ref[...]
    for r in range(_CANDS):
        m = jnp.max(ext, axis=1, keepdims=True)
        tag = m & _FMASK
        is_run = tag >= chunk
        slot = run_tag0 - tag
        localpos = (chunk - 1) - tag
        abs_run = jnp.sum(jnp.where(lane128 == slot, ri, 0),
                          axis=1, keepdims=True)
        abs_idx = jnp.where(is_run, abs_run, base + localpos)
        new_k.append((m - tag + (run_tag0 - r))[:, 0])
        new_i.append(abs_idx[:, 0])
        ext = jnp.where(ext == m, 0, ext)

    nk = jnp.stack(new_k, axis=1)
    ni = jnp.stack(new_i, axis=1)
    zpad = jnp.zeros((rows, 128 - _CANDS), jnp.int32)
    rk_ref[...] = jnp.concatenate([nk, zpad], axis=1)
    ri_ref[...] = jnp.concatenate([ni, zpad], axis=1)

    @pl.when(c == n_chunks - 1)
    def _emit():
        idx_out_ref[...] = ri_ref[...].reshape(bg, _TOPK, 128)


def _filter_indices(encoded, memory, chunk=4096, bg=4):
    B, L, D = encoded.shape
    kv_len = memory.shape[1]
    n_chunks = kv_len // chunk
    body = functools.partial(_filter_body, chunk=chunk, n_chunks=n_chunks,
                             kv_len=kv_len, bg=bg)
    return pl.pallas_call(
        body,
        grid=(B // bg, n_chunks),
        in_specs=[
            pl.BlockSpec((bg, L, D), lambda g, c: (g, 0, 0)),
            pl.BlockSpec((bg, chunk, D), lambda g, c: (g, c, 0)),
        ],
        out_specs=pl.BlockSpec((bg, L, 128), lambda g, c: (g, 0, 0)),
        out_shape=jax.ShapeDtypeStruct((B, L, 128), jnp.int32),
        scratch_shapes=[
            pltpu.VMEM((bg * _TOPK, 128), jnp.int32),
            pltpu.VMEM((bg * _TOPK, 128), jnp.int32),
        ],
    )(encoded, memory)


def _gather_rows(mem_flat, idx_flat):
    n_rows, D = mem_flat.shape
    n_idx = idx_flat.shape[0]
    info = plsc.get_sparse_core_info()
    nc, ns = info.num_cores, info.num_subcores
    nw = nc * ns
    per_w = n_idx // nw

    mesh = plsc.VectorSubcoreMesh(core_axis_name="c", subcore_axis_name="s")

    @functools.partial(
        pl.kernel, mesh=mesh,
        out_type=jax.ShapeDtypeStruct((n_idx, D), jnp.float32),
        scratch_types=[
            pltpu.VMEM((per_w,), jnp.int32),
            pltpu.VMEM((per_w, D), jnp.float32),
            pltpu.SemaphoreType.DMA,
        ],
    )
    def k(mem_hbm, idx_hbm, out_hbm, idx_v, rows_v, sem):
        wid = lax.axis_index("s") * nc + lax.axis_index("c")
        base = wid * per_w
        pltpu.sync_copy(idx_hbm.at[pl.ds(base, per_w)], idx_v)
        pltpu.async_copy(mem_hbm.at[idx_v], rows_v, sem).wait()
        pltpu.sync_copy(rows_v, out_hbm.at[pl.ds(base, per_w)])

    return k(mem_flat, idx_flat)


def _rescore_body(enc_ref, rows_ref, w_ref, b_ref, out_ref, *, nq):
    rows = rows_ref[...]
    n2 = jnp.sum(rows * rows, axis=1)
    norm = jnp.where(n2 == 0.0, 0.0, n2 * lax.rsqrt(n2))
    rown = rows / jnp.maximum(norm, 1e-12)[:, None]

    enc = enc_ref[...]
    en2 = jnp.sum(enc * enc, axis=1)
    enorm = jnp.where(en2 == 0.0, 0.0, en2 * lax.rsqrt(en2))
    encn = enc / jnp.maximum(enorm, 1e-12)[:, None]

    rb = rown.astype(jnp.bfloat16).astype(jnp.float32)
    eb = encn.astype(jnp.bfloat16).astype(jnp.float32)
    prod = rb.reshape(nq, _CANDS, 128) * eb.reshape(nq, 1, 128)
    simx = jnp.sum(prod, axis=2)

    pos = lax.broadcasted_iota(jnp.int32, (nq, _CANDS), 1)
    wsel = jnp.zeros((nq, _CANDS), jnp.float32)
    ext = simx
    for _ in range(_TOPK):
        m = jnp.max(ext, axis=1, keepdims=True)
        eq = ext == m
        first = jnp.min(jnp.where(eq, pos, _CANDS), axis=1, keepdims=True)
        onehot = pos == first
        wsel = wsel + jnp.where(onehot, 1.0, 0.0)
        ext = jnp.where(onehot, _NEG, ext)

    picked = rows.reshape(nq, _CANDS, 128) * wsel[:, :, None]
    matched = jnp.sum(picked, axis=1) * (1.0 / _TOPK)

    x = enc + matched
    y = lax.dot_general(x, w_ref[...], (((1,), (1,)), ((), ())),
                        preferred_element_type=jnp.float32)
    out_ref[...] = y + b_ref[...]


def _rescore(enc_flat, rows, W, b):
    nq, D = enc_flat.shape
    return pl.pallas_call(
        functools.partial(_rescore_body, nq=nq),
        out_shape=jax.ShapeDtypeStruct((nq, D), jnp.float32),
    )(enc_flat, rows, W, b.reshape(1, D))


def kernel(encoded, memory, W, b):
    B, L, D = encoded.shape
    kv_len = memory.shape[1]
    idx_pad = _filter_indices(encoded, memory)
    idx_flat = idx_pad[:, :, :_CANDS].reshape(-1)
    rows = _gather_rows(memory.reshape(B * kv_len, D), idx_flat)
    out = _rescore(encoded.reshape(B * L, D), rows, W, b)
    return out.reshape(B, L, D)


# speedup vs baseline: 20.1457x; 1.2500x over previous
import functools

import jax
import jax.numpy as jnp
from jax import lax
from jax.experimental import pallas as pl
from jax.experimental.pallas import tpu as pltpu
from jax.experimental.pallas import tpu_sc as plsc

_TOPK = 8
_CANDS = 12
_NEG = float("-inf")
_SHIFT = 8192
_FMASK = _SHIFT - 1


def _filter_body(enc_ref, mem_ref, idx_out_ref, rk_ref, ri_ref, *,
                 chunk, n_chunks, kv_len, bg):
    g = pl.program_id(0)
    c = pl.program_id(1)
    rows = bg * _TOPK

    @pl.when(c == 0)
    def _init():
        rk_ref[...] = jnp.zeros((rows, 128), jnp.int32)
        ri_ref[...] = jnp.zeros((rows, 128), jnp.int32)

    ones = jnp.ones((1, 128), jnp.float32)
    keys_parts = []
    for i in range(bg):
        enc = enc_ref[i]
        mem = mem_ref[i]
        en = jnp.sum(enc * enc, axis=1, keepdims=True)
        encn = enc / jnp.maximum(jnp.sqrt(en), 1e-12)
        n2 = lax.dot_general(ones, mem * mem, (((1,), (1,)), ((), ())),
                             preferred_element_type=jnp.float32)
        scale = lax.rsqrt(jnp.maximum(n2, 1e-24))
        sim = lax.dot_general(encn, mem, (((1,), (1,)), ((), ())),
                              preferred_element_type=jnp.float32)
        sim = sim * scale
        t = ((sim + 1.25) * 65536.0).astype(jnp.int32)
        local = lax.broadcasted_iota(jnp.int32, (_TOPK, chunk), 1)
        keys_parts.append(t * _SHIFT + (chunk - 1 - local))

    keys = jnp.concatenate(keys_parts, axis=0)
    ext = jnp.concatenate([rk_ref[...], keys], axis=1)
    lane128 = lax.broadcasted_iota(jnp.int32, (rows, 128), 1)
    sub = lax.broadcasted_iota(jnp.int32, (rows, 1), 0) // _TOPK
    base = (g * bg + sub) * kv_len + c * chunk
    run_tag0 = chunk + _CANDS - 1

    new_k, new_i = [], []
    ri = ri_ref[...]
    for r in range(_CANDS):
        m = jnp.max(ext, axis=1, keepdims=True)
        tag = m & _FMASK
        is_run = tag >= chunk
        slot = run_tag0 - tag
        localpos = (chunk - 1) - tag
        abs_run = jnp.sum(jnp.where(lane128 == slot, ri, 0),
                          axis=1, keepdims=True)
        abs_idx = jnp.where(is_run, abs_run, base + localpos)
        new_k.append((m - tag + (run_tag0 - r))[:, 0])
        new_i.append(abs_idx[:, 0])
        ext = jnp.where(ext == m, 0, ext)

    nk = jnp.stack(new_k, axis=1)
    ni = jnp.stack(new_i, axis=1)
    zpad = jnp.zeros((rows, 128 - _CANDS), jnp.int32)
    rk_ref[...] = jnp.concatenate([nk, zpad], axis=1)
    ri_ref[...] = jnp.concatenate([ni, zpad], axis=1)

    @pl.when(c == n_chunks - 1)
    def _emit():
        idx_out_ref[...] = ri_ref[...].reshape(bg, _TOPK, 128)


def _filter_indices(encoded, memory, chunk=4096, bg=8):
    B, L, D = encoded.shape
    kv_len = memory.shape[1]
    n_chunks = kv_len // chunk
    body = functools.partial(_filter_body, chunk=chunk, n_chunks=n_chunks,
                             kv_len=kv_len, bg=bg)
    return pl.pallas_call(
        body,
        grid=(B // bg, n_chunks),
        in_specs=[
            pl.BlockSpec((bg, L, D), lambda g, c: (g, 0, 0)),
            pl.BlockSpec((bg, chunk, D), lambda g, c: (g, c, 0)),
        ],
        out_specs=pl.BlockSpec((bg, L, 128), lambda g, c: (g, 0, 0)),
        out_shape=jax.ShapeDtypeStruct((B, L, 128), jnp.int32),
        scratch_shapes=[
            pltpu.VMEM((bg * _TOPK, 128), jnp.int32),
            pltpu.VMEM((bg * _TOPK, 128), jnp.int32),
        ],
    )(encoded, memory)


def _gather_rows(mem_flat, idx_flat):
    n_rows, D = mem_flat.shape
    n_idx = idx_flat.shape[0]
    info = plsc.get_sparse_core_info()
    nc, ns = info.num_cores, info.num_subcores
    nw = nc * ns
    per_w = n_idx // nw

    mesh = plsc.VectorSubcoreMesh(core_axis_name="c", subcore_axis_name="s")

    @functools.partial(
        pl.kernel, mesh=mesh,
        out_type=jax.ShapeDtypeStruct((n_idx, D), jnp.float32),
        scratch_types=[
            pltpu.VMEM((per_w,), jnp.int32),
            pltpu.VMEM((per_w, D), jnp.float32),
            pltpu.SemaphoreType.DMA,
        ],
    )
    def k(mem_hbm, idx_hbm, out_hbm, idx_v, rows_v, sem):
        wid = lax.axis_index("s") * nc + lax.axis_index("c")
        base = wid * per_w
        pltpu.sync_copy(idx_hbm.at[pl.ds(base, per_w)], idx_v)
        pltpu.async_copy(mem_hbm.at[idx_v], rows_v, sem).wait()
        pltpu.sync_copy(rows_v, out_hbm.at[pl.ds(base, per_w)])

    return k(mem_flat, idx_flat)


def _rescore_body(enc_ref, rows_ref, w_ref, b_ref, out_ref, *, nq):
    rows = rows_ref[...]
    n2 = jnp.sum(rows * rows, axis=1)
    norm = jnp.where(n2 == 0.0, 0.0, n2 * lax.rsqrt(n2))
    rown = rows / jnp.maximum(norm, 1e-12)[:, None]

    enc = enc_ref[...]
    en2 = jnp.sum(enc * enc, axis=1)
    enorm = jnp.where(en2 == 0.0, 0.0, en2 * lax.rsqrt(en2))
    encn = enc / jnp.maximum(enorm, 1e-12)[:, None]

    rb = rown.astype(jnp.bfloat16).astype(jnp.float32)
    eb = encn.astype(jnp.bfloat16).astype(jnp.float32)
    prod = rb.reshape(nq, _CANDS, 128) * eb.reshape(nq, 1, 128)
    simx = jnp.sum(prod, axis=2)

    pos = lax.broadcasted_iota(jnp.int32, (nq, _CANDS), 1)
    wsel = jnp.zeros((nq, _CANDS), jnp.float32)
    ext = simx
    for _ in range(_TOPK):
        m = jnp.max(ext, axis=1, keepdims=True)
        eq = ext == m
        first = jnp.min(jnp.where(eq, pos, _CANDS), axis=1, keepdims=True)
        onehot = pos == first
        wsel = wsel + jnp.where(onehot, 1.0, 0.0)
        ext = jnp.where(onehot, _NEG, ext)

    picked = rows.reshape(nq, _CANDS, 128) * wsel[:, :, None]
    matched = jnp.sum(picked, axis=1) * (1.0 / _TOPK)

    x = enc + matched
    y = lax.dot_general(x, w_ref[...], (((1,), (1,)), ((), ())),
                        preferred_element_type=jnp.float32)
    out_ref[...] = y + b_ref[...]


def _rescore(enc_flat, rows, W, b):
    nq, D = enc_flat.shape
    return pl.pallas_call(
        functools.partial(_rescore_body, nq=nq),
        out_shape=jax.ShapeDtypeStruct((nq, D), jnp.float32),
    )(enc_flat, rows, W, b.reshape(1, D))


def kernel(encoded, memory, W, b):
    B, L, D = encoded.shape
    kv_len = memory.shape[1]
    idx_pad = _filter_indices(encoded, memory)
    idx_flat = idx_pad[:, :, :_CANDS].reshape(-1)
    rows = _gather_rows(memory.reshape(B * kv_len, D), idx_flat)
    out = _rescore(encoded.reshape(B * L, D), rows, W, b)
    return out.reshape(B, L, D)


# speedup vs baseline: 20.1674x; 1.0011x over previous
import functools

import jax
import jax.numpy as jnp
from jax import lax
from jax.experimental import pallas as pl
from jax.experimental.pallas import tpu as pltpu
from jax.experimental.pallas import tpu_sc as plsc

_TOPK = 8
_CANDS = 12
_NEG = float("-inf")
_SHIFT = 8192
_FMASK = _SHIFT - 1


def _filter_body(enc_ref, mem_ref, idx_out_ref, rk_ref, ri_ref, *,
                 chunk, n_chunks, kv_len, bg):
    g = pl.program_id(0)
    c = pl.program_id(1)
    rows = bg * _TOPK

    @pl.when(c == 0)
    def _init():
        rk_ref[...] = jnp.zeros((rows, 128), jnp.int32)
        ri_ref[...] = jnp.zeros((rows, 128), jnp.int32)

    ones = jnp.ones((1, 128), jnp.float32)
    keys_parts = []
    for i in range(bg):
        enc = enc_ref[i]
        mem = mem_ref[i]
        en = jnp.sum(enc * enc, axis=1, keepdims=True)
        encn = enc / jnp.maximum(jnp.sqrt(en), 1e-12)
        n2 = lax.dot_general(ones, mem * mem, (((1,), (1,)), ((), ())),
                             preferred_element_type=jnp.float32)
        scale = lax.rsqrt(jnp.maximum(n2, 1e-24))
        sim = lax.dot_general(encn, mem, (((1,), (1,)), ((), ())),
                              preferred_element_type=jnp.float32)
        sim = sim * scale
        t = ((sim + 1.25) * 65536.0).astype(jnp.int32)
        local = lax.broadcasted_iota(jnp.int32, (_TOPK, chunk), 1)
        keys_parts.append(t * _SHIFT + (chunk - 1 - local))

    keys = jnp.concatenate(keys_parts, axis=0)
    ext = jnp.concatenate([rk_ref[...], keys], axis=1)
    lane128 = lax.broadcasted_iota(jnp.int32, (rows, 128), 1)
    sub = lax.broadcasted_iota(jnp.int32, (rows, 1), 0) // _TOPK
    base = (g * bg + sub) * kv_len + c * chunk
    run_tag0 = chunk + _CANDS - 1

    new_k, new_i = [], []
    ri = ri_ref[...]
    for r in range(_CANDS):
        m = jnp.max(ext, axis=1, keepdims=True)
        tag = m & _FMASK
        is_run = tag >= chunk
        slot = run_tag0 - tag
        localpos = (chunk - 1) - tag
        abs_run = jnp.sum(jnp.where(lane128 == slot, ri, 0),
                          axis=1, keepdims=True)
        abs_idx = jnp.where(is_run, abs_run, base + localpos)
        new_k.append((m - tag + (run_tag0 - r))[:, 0])
        new_i.append(abs_idx[:, 0])
        ext = jnp.where(ext == m, 0, ext)

    nk = jnp.stack(new_k, axis=1)
    ni = jnp.stack(new_i, axis=1)
    zpad = jnp.zeros((rows, 128 - _CANDS), jnp.int32)
    rk_ref[...] = jnp.concatenate([nk, zpad], axis=1)
    ri_ref[...] = jnp.concatenate([ni, zpad], axis=1)

    @pl.when(c == n_chunks - 1)
    def _emit():
        idx_out_ref[...] = ri_ref[...].reshape(bg, _TOPK, 128)


def _filter_indices(encoded, memory, chunk=2048, bg=16):
    B, L, D = encoded.shape
    kv_len = memory.shape[1]
    n_chunks = kv_len // chunk
    body = functools.partial(_filter_body, chunk=chunk, n_chunks=n_chunks,
                             kv_len=kv_len, bg=bg)
    return pl.pallas_call(
        body,
        grid=(B // bg, n_chunks),
        in_specs=[
            pl.BlockSpec((bg, L, D), lambda g, c: (g, 0, 0)),
            pl.BlockSpec((bg, chunk, D), lambda g, c: (g, c, 0)),
        ],
        out_specs=pl.BlockSpec((bg, L, 128), lambda g, c: (g, 0, 0)),
        out_shape=jax.ShapeDtypeStruct((B, L, 128), jnp.int32),
        scratch_shapes=[
            pltpu.VMEM((bg * _TOPK, 128), jnp.int32),
            pltpu.VMEM((bg * _TOPK, 128), jnp.int32),
        ],
    )(encoded, memory)


def _gather_rows(mem_flat, idx_flat):
    n_rows, D = mem_flat.shape
    n_idx = idx_flat.shape[0]
    info = plsc.get_sparse_core_info()
    nc, ns = info.num_cores, info.num_subcores
    nw = nc * ns
    per_w = n_idx // nw

    mesh = plsc.VectorSubcoreMesh(core_axis_name="c", subcore_axis_name="s")

    @functools.partial(
        pl.kernel, mesh=mesh,
        out_type=jax.ShapeDtypeStruct((n_idx, D), jnp.float32),
        scratch_types=[
            pltpu.VMEM((per_w,), jnp.int32),
            pltpu.VMEM((per_w, D), jnp.float32),
            pltpu.SemaphoreType.DMA,
        ],
    )
    def k(mem_hbm, idx_hbm, out_hbm, idx_v, rows_v, sem):
        wid = lax.axis_index("s") * nc + lax.axis_index("c")
        base = wid * per_w
        pltpu.sync_copy(idx_hbm.at[pl.ds(base, per_w)], idx_v)
        pltpu.async_copy(mem_hbm.at[idx_v], rows_v, sem).wait()
        pltpu.sync_copy(rows_v, out_hbm.at[pl.ds(base, per_w)])

    return k(mem_flat, idx_flat)


def _rescore_body(enc_ref, rows_ref, w_ref, b_ref, out_ref, *, nq):
    rows = rows_ref[...]
    n2 = jnp.sum(rows * rows, axis=1)
    norm = jnp.where(n2 == 0.0, 0.0, n2 * lax.rsqrt(n2))
    rown = rows / jnp.maximum(norm, 1e-12)[:, None]

    enc = enc_ref[...]
    en2 = jnp.sum(enc * enc, axis=1)
    enorm = jnp.where(en2 == 0.0, 0.0, en2 * lax.rsqrt(en2))
    encn = enc / jnp.maximum(enorm, 1e-12)[:, None]

    rb = rown.astype(jnp.bfloat16).astype(jnp.float32)
    eb = encn.astype(jnp.bfloat16).astype(jnp.float32)
    prod = rb.reshape(nq, _CANDS, 128) * eb.reshape(nq, 1, 128)
    simx = jnp.sum(prod, axis=2)

    pos = lax.broadcasted_iota(jnp.int32, (nq, _CANDS), 1)
    wsel = jnp.zeros((nq, _CANDS), jnp.float32)
    ext = simx
    for _ in range(_TOPK):
        m = jnp.max(ext, axis=1, keepdims=True)
        eq = ext == m
        first = jnp.min(jnp.where(eq, pos, _CANDS), axis=1, keepdims=True)
        onehot = pos == first
        wsel = wsel + jnp.where(onehot, 1.0, 0.0)
        ext = jnp.where(onehot, _NEG, ext)

    picked = rows.reshape(nq, _CANDS, 128) * wsel[:, :, None]
    matched = jnp.sum(picked, axis=1) * (1.0 / _TOPK)

    x = enc + matched
    y = lax.dot_general(x, w_ref[...], (((1,), (1,)), ((), ())),
                        preferred_element_type=jnp.float32)
    out_ref[...] = y + b_ref[...]


def _rescore(enc_flat, rows, W, b):
    nq, D = enc_flat.shape
    return pl.pallas_call(
        functools.partial(_rescore_body, nq=nq),
        out_shape=jax.ShapeDtypeStruct((nq, D), jnp.float32),
    )(enc_flat, rows, W, b.reshape(1, D))


def kernel(encoded, memory, W, b):
    B, L, D = encoded.shape
    kv_len = memory.shape[1]
    idx_pad = _filter_indices(encoded, memory)
    idx_flat = idx_pad[:, :, :_CANDS].reshape(-1)
    rows = _gather_rows(memory.reshape(B * kv_len, D), idx_flat)
    out = _rescore(encoded.reshape(B * L, D), rows, W, b)
    return out.reshape(B, L, D)
